# Initial kernel scaffold; baseline (speedup 1.0000x reference)
import jax
import jax.numpy as jnp
from jax.experimental import pallas as pl


def kernel(weight_probs, weight_indices, bias_probs, bias_indices, x, input_weight_bank, output_weight_bank, diagonal_weight_bank, anti_diagonal_weight_bank, bias_bank):
    raise NotImplementedError("write your pallas kernel here")


# speedup vs baseline: 5.4306x; 5.4306x over previous
import functools

import jax
import jax.numpy as jnp
from jax import lax
from jax.experimental import pallas as pl
from jax.experimental.pallas import tpu as pltpu


def _stage_a(x, banks, interpret=False):
    B, D = x.shape
    E = banks[0].shape[0]
    NB = len(banks)

    def body(x_ref, *refs):
        out_ref = refs[-1]
        xv = x_ref[...]
        for t in range(NB):
            out_ref[0, t, :, :] = jnp.dot(xv, refs[t][0],
                                          preferred_element_type=jnp.float32)

    return pl.pallas_call(
        body,
        grid=(E,),
        in_specs=[pl.BlockSpec((B, D), lambda e: (0, 0))]
        + [pl.BlockSpec((1, D, D), lambda e: (e, 0, 0)) for _ in range(NB)],
        out_specs=pl.BlockSpec((1, NB, B, D), lambda e: (e, 0, 0, 0)),
        out_shape=jax.ShapeDtypeStruct((E, NB, B, D), jnp.float32),
        interpret=interpret,
    )(x, *banks)


def _stage_c(allY, widx, bidx, wp, bp, interpret=False):
    E, NB, B, D = allY.shape

    def body(widx_ref, bidx_ref, wp_ref, bp_ref, y_ref, wm_ref, bias_ref):
        b = pl.program_id(0)
        i0 = widx_ref[b, 0]
        i1 = widx_ref[b, 1]
        j0 = bidx_ref[b, 0]
        j1 = bidx_ref[b, 1]
        wp0 = wp_ref[b, 0]
        wp1 = wp_ref[b, 1]
        bp0 = bp_ref[b, 0]
        bp1 = bp_ref[b, 1]

        def row(e, t):
            return y_ref[pl.ds(e, 1), t, pl.ds(b, 1), :].reshape(1, D)

        u0 = row(i0, 0) * wp0
        u1 = row(i1, 0) * wp1
        v0 = row(i0, 1)
        v1 = row(i1, 1)
        d0 = row(i0, 2) * wp0
        d1 = row(i1, 2) * wp1
        a0 = row(i0, 3) * wp0
        a1 = row(i1, 3) * wp1
        c0 = row(j0, 4) * bp0
        c1 = row(j1, 4) * bp1

        bias_ref[0] = c0 + c1

        Uw = jnp.concatenate([u0, u1], axis=0)
        V = jnp.concatenate([v0, v1], axis=0)
        cdims = (((0,), (0,)), ((), ()))
        outer = lax.dot_general(Uw, V, cdims,
                                preferred_element_type=jnp.float32)
        m = jnp.mean(V, axis=1, keepdims=True)
        mu = lax.dot_general(Uw, m, cdims,
                             preferred_element_type=jnp.float32)
        G = lax.dot_general(V, V, (((1,), (1,)), ((), ())),
                            preferred_element_type=jnp.float32) / D
        W2 = lax.dot_general(G, Uw, (((1,), (0,)), ((), ())),
                             preferred_element_type=jnp.float32)
        ones = jnp.ones((2, 1), jnp.float32)
        ex2 = lax.dot_general(Uw * W2, ones, cdims,
                              preferred_element_type=jnp.float32)
        var = ex2 - mu * mu
        rs = lax.rsqrt(var + 1e-5)
        out = (outer - mu) * rs

        rows = lax.broadcasted_iota(jnp.int32, (D, D), 0)
        cols = lax.broadcasted_iota(jnp.int32, (D, D), 1)
        dmix = d0 + d1
        amix_col = lax.dot_general(
            jnp.concatenate([a0, a1], axis=0), ones, cdims,
            preferred_element_type=jnp.float32)
        out = out + jnp.where(rows == cols, dmix, 0.0)
        out = out + jnp.where(rows + cols == D - 1, amix_col, 0.0)
        wm_ref[0] = out

    smem = functools.partial(pl.BlockSpec, memory_space=pltpu.SMEM)
    return pl.pallas_call(
        body,
        grid=(B,),
        in_specs=[smem(), smem(), smem(), smem(),
                  pl.BlockSpec((E, NB, B, D), lambda b: (0, 0, 0, 0))],
        out_specs=[pl.BlockSpec((1, D, D), lambda b: (b, 0, 0)),
                   pl.BlockSpec((1, 1, D), lambda b: (b, 0, 0))],
        out_shape=[jax.ShapeDtypeStruct((B, D, D), jnp.float32),
                   jax.ShapeDtypeStruct((B, 1, D), jnp.float32)],
        interpret=interpret,
    )(widx, bidx, wp, bp, allY)


def kernel(weight_probs, weight_indices, bias_probs, bias_indices, x,
           input_weight_bank, output_weight_bank, diagonal_weight_bank,
           anti_diagonal_weight_bank, bias_bank, interpret=False):
    widx = weight_indices.astype(jnp.int32)
    bidx = bias_indices.astype(jnp.int32)
    banks = (input_weight_bank, output_weight_bank, diagonal_weight_bank,
             anti_diagonal_weight_bank, bias_bank)
    allY = _stage_a(x, banks, interpret=interpret)
    wm, bias = _stage_c(allY, widx, bidx, weight_probs, bias_probs,
                        interpret=interpret)
    return wm, bias.reshape(bias.shape[0], bias.shape[2])
